# baseline (speedup 1.0000x reference)
import jax
import jax.numpy as jnp
from jax.experimental import pallas as pl
from jax.experimental.pallas import tpu as pltpu

_Q = 8


def _rowdot_kernel(x_hbm, out_ref, *rest):
    bufs = rest[: 2 * _Q]
    sems = rest[2 * _Q :]
    n = out_ref.shape[0]
    qcols = n // _Q
    copies = []
    for q in range(_Q):
        a = pltpu.make_async_copy(
            x_hbm.at[0, :, pl.ds(q * qcols, qcols)], bufs[2 * q], sems[q]
        )
        b = pltpu.make_async_copy(
            x_hbm.at[1, :, pl.ds(q * qcols, qcols)], bufs[2 * q + 1], sems[q]
        )
        a.start()
        b.start()
        copies.append((a, b))
    for q in range(_Q):
        a, b = copies[q]
        a.wait()
        b.wait()
        out_ref[pl.ds(q * qcols, qcols)] = jnp.sum(
            bufs[2 * q][...] * bufs[2 * q + 1][...], axis=0
        )


def kernel(inputs):
    n = inputs.shape[1]
    d = inputs.shape[2]
    t = jnp.swapaxes(inputs, 1, 2)
    qcols = n // _Q
    return pl.pallas_call(
        _rowdot_kernel,
        in_specs=[pl.BlockSpec(memory_space=pltpu.MemorySpace.HBM)],
        out_specs=pl.BlockSpec(memory_space=pltpu.MemorySpace.VMEM),
        out_shape=jax.ShapeDtypeStruct((n,), inputs.dtype),
        scratch_shapes=(
            [pltpu.VMEM((d, qcols), jnp.float32) for _ in range(2 * _Q)]
            + [pltpu.SemaphoreType.DMA for _ in range(_Q)]
        ),
    )(t)
